# Initial kernel scaffold; baseline (speedup 1.0000x reference)
import jax
import jax.numpy as jnp
from jax.experimental import pallas as pl


def kernel(x, edge_index, W, b, Wr, br):
    raise NotImplementedError("write your pallas kernel here")


# speedup vs baseline: 14.0021x; 14.0021x over previous
import functools

import jax
import jax.numpy as jnp
from jax import lax
from jax.experimental import pallas as pl
from jax.experimental.pallas import tpu as pltpu
from jax.experimental.pallas import tpu_sc as plsc

N = 10000
D = 128
E = 320000
NHOPS = 3

NC = 2
NS = 16
NW = NC * NS

NPAD = 10240
EB = 128
NB = 79
EPT = NB * EB
EPAD = NW * EPT
RPT = NPAD // NS

_mesh = plsc.VectorSubcoreMesh(core_axis_name="c", subcore_axis_name="s")


def _zero_rows(ref, nrows, ncols):
    def row(i, c):
        def lane(k, c2):
            ref[i, pl.ds(k * 16, 16)] = jnp.zeros((16,), jnp.float32)
            return c2
        return lax.fori_loop(0, ncols // 16, lane, c)
    lax.fori_loop(0, nrows, row, 0)


@functools.partial(
    pl.kernel,
    out_type=jax.ShapeDtypeStruct((NC, NPAD, D), jnp.float32),
    mesh=_mesh,
    scratch_types=[
        pltpu.VMEM((NB, EB), jnp.int32),
        pltpu.VMEM((EB, D), jnp.float32),
        pltpu.VMEM_SHARED((NPAD, D), jnp.float32),
    ],
)
def _deg_call(row_hbm, out_hbm, row_v, ones_v, acc):
    cid = lax.axis_index("c")
    sid = lax.axis_index("s")
    wid = cid * NS + sid

    pltpu.sync_copy(row_hbm.at[wid], row_v)

    _zero_rows(ones_v, EB, D)
    for j in range(RPT // EB):
        pltpu.sync_copy(ones_v, acc.at[pl.ds(sid * RPT + j * EB, EB)])

    def fill(i, c):
        ones_v[i, pl.ds(0, 16)] = jnp.ones((16,), jnp.float32)
        return c
    lax.fori_loop(0, EB, fill, 0)
    plsc.subcore_barrier()

    def body(j, c):
        pltpu.sync_copy(ones_v, acc.at[row_v.at[j]], add=True)
        return c
    lax.fori_loop(0, NB, body, 0)

    plsc.subcore_barrier()
    pltpu.sync_copy(acc.at[pl.ds(sid * RPT, RPT)],
                    out_hbm.at[cid, pl.ds(sid * RPT, RPT)])


@functools.partial(
    pl.kernel,
    out_type=jax.ShapeDtypeStruct((NC, NPAD, D), jnp.float32),
    mesh=_mesh,
    scratch_types=[
        pltpu.VMEM((NB, EB), jnp.int32),
        pltpu.VMEM((NB, EB), jnp.int32),
        pltpu.VMEM((EB, D), jnp.float32),
        pltpu.VMEM_SHARED((NPAD, D), jnp.float32),
        pltpu.SemaphoreType.DMA,
    ],
)
def _hop_call(row_hbm, col_hbm, g_hbm, out_hbm, row_v, col_v, buf, acc, sem):
    cid = lax.axis_index("c")
    sid = lax.axis_index("s")
    wid = cid * NS + sid

    pltpu.sync_copy(row_hbm.at[wid], row_v)
    pltpu.sync_copy(col_hbm.at[wid], col_v)

    _zero_rows(buf, EB, D)
    for j in range(RPT // EB):
        pltpu.sync_copy(buf, acc.at[pl.ds(sid * RPT + j * EB, EB)])
    plsc.subcore_barrier()

    def body(j, c):
        pltpu.async_copy(g_hbm.at[col_v.at[j]], buf, sem).wait()
        pltpu.sync_copy(buf, acc.at[row_v.at[j]], add=True)
        return c
    lax.fori_loop(0, NB, body, 0)

    plsc.subcore_barrier()
    pltpu.sync_copy(acc.at[pl.ds(sid * RPT, RPT)],
                    out_hbm.at[cid, pl.ds(sid * RPT, RPT)])


_TB = 512
_GRID = NPAD // _TB


def _dinv_block(degp):
    deg = degp[0] + degp[1]
    return jnp.where(deg > 0.0, lax.rsqrt(deg), 0.0)[:, 0:1]


def _scale0_body(degp_ref, x_ref, g_ref):
    g_ref[...] = x_ref[...] * _dinv_block(degp_ref[...])


def _scaleh_body(degp_ref, p_ref, h_ref, g_ref):
    dinv = _dinv_block(degp_ref[...])
    h = (p_ref[0] + p_ref[1]) * dinv
    h_ref[...] = h
    g_ref[...] = h * dinv


def _final_body(degp_ref, x_ref, h1_ref, h2_ref, p3_ref, W_ref, b_ref,
                Wr_ref, br_ref, o_ref):
    dinv = _dinv_block(degp_ref[...])
    h3 = (p3_ref[0] + p3_ref[1]) * dinv
    feats = (x_ref[...], h1_ref[...], h2_ref[...], h3)
    acc = jnp.zeros((_TB, D), jnp.float32)
    for i in range(NHOPS + 1):
        mm = jnp.dot(feats[i], W_ref[i], preferred_element_type=jnp.float32)
        acc = acc + jnp.maximum(mm + b_ref[i][None, :], 0.0)
    out = jnp.dot(acc, Wr_ref[...], preferred_element_type=jnp.float32)
    o_ref[...] = jnp.maximum(out + br_ref[0][None, :], 0.0)


_degp_spec = pl.BlockSpec((NC, _TB, D), lambda i: (0, i, 0))
_node_spec = pl.BlockSpec((_TB, D), lambda i: (i, 0))
_part_spec = pl.BlockSpec((NC, _TB, D), lambda i: (0, i, 0))
_full = lambda shape: pl.BlockSpec(shape, lambda i: tuple(0 for _ in shape))

_scale0 = pl.pallas_call(
    _scale0_body,
    grid=(_GRID,),
    in_specs=[_degp_spec, _node_spec],
    out_specs=_node_spec,
    out_shape=jax.ShapeDtypeStruct((NPAD, D), jnp.float32),
)

_scaleh = pl.pallas_call(
    _scaleh_body,
    grid=(_GRID,),
    in_specs=[_degp_spec, _part_spec],
    out_specs=(_node_spec, _node_spec),
    out_shape=(jax.ShapeDtypeStruct((NPAD, D), jnp.float32),
               jax.ShapeDtypeStruct((NPAD, D), jnp.float32)),
)

_final = pl.pallas_call(
    _final_body,
    grid=(_GRID,),
    in_specs=[_degp_spec, _node_spec, _node_spec, _node_spec, _part_spec,
              _full((NHOPS + 1, D, D)), _full((8, D)),
              _full((D, D)), _full((8, D))],
    out_specs=_node_spec,
    out_shape=jax.ShapeDtypeStruct((NPAD, D), jnp.float32),
)


def kernel(x, edge_index, W, b, Wr, br):
    row = edge_index[0]
    col = edge_index[1]

    pad = EPAD - E
    ar = jnp.arange(pad, dtype=jnp.int32)
    rowp = jnp.concatenate([row, N + ar % (NPAD - N)]).reshape(NW, NB, EB)
    colp = jnp.concatenate([col, ar % N]).reshape(NW, NB, EB)

    xp = jnp.pad(x, ((0, NPAD - N), (0, 0)))
    b8 = jnp.pad(b, ((0, 8 - (NHOPS + 1)), (0, 0)))
    br8 = jnp.pad(br[None, :], ((0, 7), (0, 0)))

    degp = _deg_call(rowp)
    g = _scale0(degp, xp)
    feats = []
    for hop in range(NHOPS):
        p = _hop_call(rowp, colp, g)
        if hop < NHOPS - 1:
            h, g = _scaleh(degp, p)
            feats.append(h)
    out = _final(degp, xp, feats[0], feats[1], p, W, b8, Wr, br8)
    return out[:N]


# speedup vs baseline: 19.7503x; 1.4105x over previous
import functools

import jax
import jax.numpy as jnp
from jax import lax
from jax.experimental import pallas as pl
from jax.experimental.pallas import tpu as pltpu
from jax.experimental.pallas import tpu_sc as plsc

N = 10000
D = 128
E = 320000
NHOPS = 3

NC = 2
NS = 16
NW = NC * NS

NPAD = 10240
NACC = 10112
EB = 128
NB = 80
NBUF = 2
NGRP = NB // NBUF
DEGBUF = 4
EPT = NB * EB
EPAD = NW * EPT
RPTA = NACC // NS
PSH = 14

_mesh = plsc.VectorSubcoreMesh(core_axis_name="c", subcore_axis_name="s")


def _zero_rows(ref, nrows, ncols):
    def row(i, c):
        def lane(k, c2):
            ref[i, pl.ds(k * 16, 16)] = jnp.zeros((16,), jnp.float32)
            return c2
        return lax.fori_loop(0, ncols // 16, lane, c)
    lax.fori_loop(0, nrows, row, 0)


def _zero_acc_slab(zbuf, acc, sid):
    base = sid * RPTA
    nfull = RPTA // EB
    for j in range(nfull):
        pltpu.sync_copy(zbuf, acc.at[pl.ds(base + j * EB, EB)])
    rem = RPTA - nfull * EB
    if rem:
        pltpu.sync_copy(zbuf.at[pl.ds(0, rem)],
                        acc.at[pl.ds(base + nfull * EB, rem)])


@functools.partial(
    pl.kernel,
    out_type=jax.ShapeDtypeStruct((NC, NPAD, D), jnp.float32),
    mesh=_mesh,
    scratch_types=[
        pltpu.VMEM((NB, EB), jnp.int32),
        pltpu.VMEM((EB, D), jnp.float32),
        pltpu.VMEM_SHARED((NACC, D), jnp.float32),
        pltpu.SemaphoreType.DMA,
        pltpu.SemaphoreType.DMA,
        pltpu.SemaphoreType.DMA,
        pltpu.SemaphoreType.DMA,
    ],
)
def _deg_call(row_hbm, out_hbm, row_v, ones_v, acc, sem0, sem1, sem2, sem3):
    cid = lax.axis_index("c")
    sid = lax.axis_index("s")
    wid = cid * NS + sid
    sems = (sem0, sem1, sem2, sem3)

    pltpu.sync_copy(row_hbm.at[wid], row_v)

    _zero_rows(ones_v, EB, D)
    _zero_acc_slab(ones_v, acc, sid)

    def fill(i, c):
        ones_v[i, pl.ds(0, 16)] = jnp.ones((16,), jnp.float32)
        return c
    lax.fori_loop(0, EB, fill, 0)
    plsc.subcore_barrier()

    for b in range(DEGBUF):
        pltpu.async_copy(ones_v, acc.at[row_v.at[b]], sems[b], add=True)

    def body(t, c):
        base = t * DEGBUF
        for b in range(DEGBUF):
            pltpu.make_async_copy(ones_v, acc.at[row_v.at[base - DEGBUF + b]],
                                  sems[b]).wait()
            pltpu.async_copy(ones_v, acc.at[row_v.at[base + b]], sems[b],
                             add=True)
        return c
    lax.fori_loop(1, NB // DEGBUF, body, 0)
    for b in range(DEGBUF):
        pltpu.make_async_copy(ones_v, acc.at[row_v.at[NB - DEGBUF + b]],
                              sems[b]).wait()

    plsc.subcore_barrier()
    pltpu.sync_copy(acc.at[pl.ds(sid * RPTA, RPTA)],
                    out_hbm.at[cid, pl.ds(sid * RPTA, RPTA)])


def _unpack(pk_v, j, dst, b, shift, mask):
    for k in range(EB // 16):
        v = pk_v[j, pl.ds(k * 16, 16)]
        if shift:
            v = lax.shift_right_logical(v, jnp.int32(shift))
        if mask is not None:
            v = lax.bitwise_and(v, jnp.int32(mask))
        dst[b, pl.ds(k * 16, 16)] = v


@functools.partial(
    pl.kernel,
    out_type=jax.ShapeDtypeStruct((NC, NPAD, D), jnp.float32),
    mesh=_mesh,
    scratch_types=[
        pltpu.VMEM((NB, EB), jnp.int32),
        pltpu.VMEM((NBUF, EB), jnp.int32),
        pltpu.VMEM((NBUF, EB), jnp.int32),
        pltpu.VMEM((EB, D), jnp.float32),
        pltpu.VMEM((EB, D), jnp.float32),
        pltpu.VMEM_SHARED((NACC, D), jnp.float32),
        pltpu.SemaphoreType.DMA,
        pltpu.SemaphoreType.DMA,
    ],
)
def _hop_call(pk_hbm, g_hbm, out_hbm, pk_v, idxr, idxc,
              buf0, buf1, acc, gs0, gs1):
    cid = lax.axis_index("c")
    sid = lax.axis_index("s")
    wid = cid * NS + sid
    bufs = (buf0, buf1)
    gsems = (gs0, gs1)

    pltpu.sync_copy(pk_hbm.at[wid], pk_v)

    _zero_rows(buf0, EB, D)
    _zero_acc_slab(buf0, acc, sid)
    plsc.subcore_barrier()

    for b in range(NBUF):
        _unpack(pk_v, b, idxc, b, 0, (1 << PSH) - 1)
        pltpu.async_copy(g_hbm.at[idxc.at[b]], bufs[b], gsems[b])

    def group(t, c):
        base = t * NBUF
        for b in range(NBUF):
            jj = base + b
            _unpack(pk_v, jj, idxr, b, PSH, None)
            pltpu.make_async_copy(g_hbm.at[idxc.at[b]], bufs[b],
                                  gsems[b]).wait()
            pltpu.sync_copy(bufs[b], acc.at[idxr.at[b]], add=True)

            @pl.when(jj + NBUF < NB)
            def _():
                _unpack(pk_v, jj + NBUF, idxc, b, 0, (1 << PSH) - 1)
                pltpu.async_copy(g_hbm.at[idxc.at[b]], bufs[b], gsems[b])
        return c
    lax.fori_loop(0, NGRP, group, 0)

    plsc.subcore_barrier()
    pltpu.sync_copy(acc.at[pl.ds(sid * RPTA, RPTA)],
                    out_hbm.at[cid, pl.ds(sid * RPTA, RPTA)])


_TB = 512
_GRID = NPAD // _TB


def _dinv_block(degp):
    deg = degp[0] + degp[1]
    return jnp.where(deg > 0.0, lax.rsqrt(deg), 0.0)[:, 0:1]


def _scale0_body(degp_ref, x_ref, g_ref):
    g_ref[...] = x_ref[...] * _dinv_block(degp_ref[...])


def _scaleh_body(degp_ref, p_ref, h_ref, g_ref):
    dinv = _dinv_block(degp_ref[...])
    h = (p_ref[0] + p_ref[1]) * dinv
    h_ref[...] = h
    g_ref[...] = h * dinv


def _final_body(degp_ref, x_ref, h1_ref, h2_ref, p3_ref, W_ref, b_ref,
                Wr_ref, br_ref, o_ref):
    dinv = _dinv_block(degp_ref[...])
    h3 = (p3_ref[0] + p3_ref[1]) * dinv
    feats = (x_ref[...], h1_ref[...], h2_ref[...], h3)
    acc = jnp.zeros((_TB, D), jnp.float32)
    for i in range(NHOPS + 1):
        mm = jnp.dot(feats[i], W_ref[i], preferred_element_type=jnp.float32)
        acc = acc + jnp.maximum(mm + b_ref[i][None, :], 0.0)
    out = jnp.dot(acc, Wr_ref[...], preferred_element_type=jnp.float32)
    o_ref[...] = jnp.maximum(out + br_ref[0][None, :], 0.0)


_degp_spec = pl.BlockSpec((NC, _TB, D), lambda i: (0, i, 0))
_node_spec = pl.BlockSpec((_TB, D), lambda i: (i, 0))
_part_spec = pl.BlockSpec((NC, _TB, D), lambda i: (0, i, 0))
_full = lambda shape: pl.BlockSpec(shape, lambda i: tuple(0 for _ in shape))

_scale0 = pl.pallas_call(
    _scale0_body,
    grid=(_GRID,),
    in_specs=[_degp_spec, _node_spec],
    out_specs=_node_spec,
    out_shape=jax.ShapeDtypeStruct((NPAD, D), jnp.float32),
)

_scaleh = pl.pallas_call(
    _scaleh_body,
    grid=(_GRID,),
    in_specs=[_degp_spec, _part_spec],
    out_specs=(_node_spec, _node_spec),
    out_shape=(jax.ShapeDtypeStruct((NPAD, D), jnp.float32),
               jax.ShapeDtypeStruct((NPAD, D), jnp.float32)),
)

_final = pl.pallas_call(
    _final_body,
    grid=(_GRID,),
    in_specs=[_degp_spec, _node_spec, _node_spec, _node_spec, _part_spec,
              _full((NHOPS + 1, D, D)), _full((8, D)),
              _full((D, D)), _full((8, D))],
    out_specs=_node_spec,
    out_shape=jax.ShapeDtypeStruct((NPAD, D), jnp.float32),
)


def kernel(x, edge_index, W, b, Wr, br):
    row = edge_index[0]
    col = edge_index[1]

    pad = EPAD - E
    ar = jnp.arange(pad, dtype=jnp.int32)
    rowp = jnp.concatenate([row, N + ar % (NACC - N)]).reshape(NW, NB, EB)
    colp = jnp.concatenate([col, ar % N]).reshape(NW, NB, EB)
    pk = (rowp << PSH) | colp

    xp = jnp.pad(x, ((0, NPAD - N), (0, 0)))
    b8 = jnp.pad(b, ((0, 8 - (NHOPS + 1)), (0, 0)))
    br8 = jnp.pad(br[None, :], ((0, 7), (0, 0)))

    degp = _deg_call(rowp)
    g = _scale0(degp, xp)
    feats = []
    for hop in range(NHOPS):
        p = _hop_call(pk, g)
        if hop < NHOPS - 1:
            h, g = _scaleh(degp, p)
            feats.append(h)
    out = _final(degp, xp, feats[0], feats[1], p, W, b8, Wr, br8)
    return out[:N]


# speedup vs baseline: 20.6050x; 1.0433x over previous
import functools

import jax
import jax.numpy as jnp
from jax import lax
from jax.experimental import pallas as pl
from jax.experimental.pallas import tpu as pltpu
from jax.experimental.pallas import tpu_sc as plsc

N = 10000
D = 128
E = 320000
NHOPS = 3

NC = 2
NS = 16
NW = NC * NS

NPAD = 10240
NACC = 10112
EB = 80
NB = 126
NBUF = 3
NGRP = NB // NBUF
DEGBUF = 3
EPT = NB * EB
EPAD = NW * EPT
RPTA = NACC // NS
PSH = 14

_mesh = plsc.VectorSubcoreMesh(core_axis_name="c", subcore_axis_name="s")


def _zero_rows(ref, nrows, ncols):
    def row(i, c):
        def lane(k, c2):
            ref[i, pl.ds(k * 16, 16)] = jnp.zeros((16,), jnp.float32)
            return c2
        return lax.fori_loop(0, ncols // 16, lane, c)
    lax.fori_loop(0, nrows, row, 0)


def _zero_acc_slab(zbuf, acc, sid):
    base = sid * RPTA
    nfull = RPTA // EB
    for j in range(nfull):
        pltpu.sync_copy(zbuf, acc.at[pl.ds(base + j * EB, EB)])
    rem = RPTA - nfull * EB
    if rem:
        pltpu.sync_copy(zbuf.at[pl.ds(0, rem)],
                        acc.at[pl.ds(base + nfull * EB, rem)])


@functools.partial(
    pl.kernel,
    out_type=jax.ShapeDtypeStruct((NC, NPAD, D), jnp.float32),
    mesh=_mesh,
    scratch_types=[
        pltpu.VMEM((NB, EB), jnp.int32),
        pltpu.VMEM((EB, D), jnp.float32),
        pltpu.VMEM_SHARED((NACC, D), jnp.float32),
        pltpu.SemaphoreType.DMA,
        pltpu.SemaphoreType.DMA,
        pltpu.SemaphoreType.DMA,
        pltpu.SemaphoreType.DMA,
    ],
)
def _deg_call(row_hbm, out_hbm, row_v, ones_v, acc, sem0, sem1, sem2, sem3):
    cid = lax.axis_index("c")
    sid = lax.axis_index("s")
    wid = cid * NS + sid
    sems = (sem0, sem1, sem2, sem3)

    pltpu.sync_copy(row_hbm.at[wid], row_v)

    _zero_rows(ones_v, EB, D)
    _zero_acc_slab(ones_v, acc, sid)

    def fill(i, c):
        ones_v[i, pl.ds(0, 16)] = jnp.ones((16,), jnp.float32)
        return c
    lax.fori_loop(0, EB, fill, 0)
    plsc.subcore_barrier()

    for b in range(DEGBUF):
        pltpu.async_copy(ones_v, acc.at[row_v.at[b]], sems[b], add=True)

    def body(t, c):
        base = t * DEGBUF
        for b in range(DEGBUF):
            pltpu.make_async_copy(ones_v, acc.at[row_v.at[base - DEGBUF + b]],
                                  sems[b]).wait()
            pltpu.async_copy(ones_v, acc.at[row_v.at[base + b]], sems[b],
                             add=True)
        return c
    lax.fori_loop(1, NB // DEGBUF, body, 0)
    for b in range(DEGBUF):
        pltpu.make_async_copy(ones_v, acc.at[row_v.at[NB - DEGBUF + b]],
                              sems[b]).wait()

    plsc.subcore_barrier()
    pltpu.sync_copy(acc.at[pl.ds(sid * RPTA, RPTA)],
                    out_hbm.at[cid, pl.ds(sid * RPTA, RPTA)])


def _unpack(pk_v, j, dst, b, shift, mask):
    for k in range(EB // 16):
        v = pk_v[j, pl.ds(k * 16, 16)]
        if shift:
            v = lax.shift_right_logical(v, jnp.int32(shift))
        if mask is not None:
            v = lax.bitwise_and(v, jnp.int32(mask))
        dst[b, pl.ds(k * 16, 16)] = v


@functools.partial(
    pl.kernel,
    out_type=jax.ShapeDtypeStruct((NC, NPAD, D), jnp.float32),
    mesh=_mesh,
    scratch_types=[
        pltpu.VMEM((NB, EB), jnp.int32),
        pltpu.VMEM((NBUF, EB), jnp.int32),
        pltpu.VMEM((NBUF, EB), jnp.int32),
        pltpu.VMEM((EB, D), jnp.float32),
        pltpu.VMEM((EB, D), jnp.float32),
        pltpu.VMEM((EB, D), jnp.float32),
        pltpu.VMEM_SHARED((NACC, D), jnp.float32),
        pltpu.SemaphoreType.DMA,
        pltpu.SemaphoreType.DMA,
        pltpu.SemaphoreType.DMA,
        pltpu.SemaphoreType.DMA,
        pltpu.SemaphoreType.DMA,
        pltpu.SemaphoreType.DMA,
    ],
)
def _hop_call(pk_hbm, g_hbm, out_hbm, pk_v, idxr, idxc,
              buf0, buf1, buf2, acc, gs0, gs1, gs2, ss0, ss1, ss2):
    cid = lax.axis_index("c")
    sid = lax.axis_index("s")
    wid = cid * NS + sid
    bufs = (buf0, buf1, buf2)
    gsems = (gs0, gs1, gs2)
    ssems = (ss0, ss1, ss2)

    pltpu.sync_copy(pk_hbm.at[wid], pk_v)

    _zero_rows(buf0, EB, D)
    _zero_acc_slab(buf0, acc, sid)
    plsc.subcore_barrier()

    for b in range(NBUF):
        _unpack(pk_v, b, idxc, b, 0, (1 << PSH) - 1)
        pltpu.async_copy(g_hbm.at[idxc.at[b]], bufs[b], gsems[b])

    def group(t, c):
        base = t * NBUF
        for b in range(NBUF):
            jj = base + b
            pltpu.make_async_copy(g_hbm.at[idxc.at[b]], bufs[b],
                                  gsems[b]).wait()
            _unpack(pk_v, jj, idxr, b, PSH, None)
            pltpu.async_copy(bufs[b], acc.at[idxr.at[b]], ssems[b], add=True)

            bf = (b + NBUF - 1) % NBUF
            jf = jj + NBUF - 1

            @pl.when((jf < NB) & (jf >= NBUF))
            def _():
                pltpu.make_async_copy(bufs[bf], acc.at[idxr.at[bf]],
                                      ssems[bf]).wait()
                _unpack(pk_v, jf, idxc, bf, 0, (1 << PSH) - 1)
                pltpu.async_copy(g_hbm.at[idxc.at[bf]], bufs[bf], gsems[bf])
        return c
    lax.fori_loop(0, NGRP, group, 0)
    for m in (NB - 3, NB - 2, NB - 1):
        b = m % NBUF
        pltpu.make_async_copy(bufs[b], acc.at[idxr.at[b]], ssems[b]).wait()

    plsc.subcore_barrier()
    pltpu.sync_copy(acc.at[pl.ds(sid * RPTA, RPTA)],
                    out_hbm.at[cid, pl.ds(sid * RPTA, RPTA)])


_TB = 512
_GRID = NPAD // _TB


def _dinv_block(degp):
    deg = degp[0] + degp[1]
    return jnp.where(deg > 0.0, lax.rsqrt(deg), 0.0)[:, 0:1]


def _scale0_body(degp_ref, x_ref, g_ref):
    g_ref[...] = x_ref[...] * _dinv_block(degp_ref[...])


def _scaleh_body(degp_ref, p_ref, h_ref, g_ref):
    dinv = _dinv_block(degp_ref[...])
    h = (p_ref[0] + p_ref[1]) * dinv
    h_ref[...] = h
    g_ref[...] = h * dinv


def _final_body(degp_ref, x_ref, h1_ref, h2_ref, p3_ref, W_ref, b_ref,
                Wr_ref, br_ref, o_ref):
    dinv = _dinv_block(degp_ref[...])
    h3 = (p3_ref[0] + p3_ref[1]) * dinv
    feats = (x_ref[...], h1_ref[...], h2_ref[...], h3)
    acc = jnp.zeros((_TB, D), jnp.float32)
    for i in range(NHOPS + 1):
        mm = jnp.dot(feats[i], W_ref[i], preferred_element_type=jnp.float32)
        acc = acc + jnp.maximum(mm + b_ref[i][None, :], 0.0)
    out = jnp.dot(acc, Wr_ref[...], preferred_element_type=jnp.float32)
    o_ref[...] = jnp.maximum(out + br_ref[0][None, :], 0.0)


_degp_spec = pl.BlockSpec((NC, _TB, D), lambda i: (0, i, 0))
_node_spec = pl.BlockSpec((_TB, D), lambda i: (i, 0))
_part_spec = pl.BlockSpec((NC, _TB, D), lambda i: (0, i, 0))
_full = lambda shape: pl.BlockSpec(shape, lambda i: tuple(0 for _ in shape))

_scale0 = pl.pallas_call(
    _scale0_body,
    grid=(_GRID,),
    in_specs=[_degp_spec, _node_spec],
    out_specs=_node_spec,
    out_shape=jax.ShapeDtypeStruct((NPAD, D), jnp.float32),
)

_scaleh = pl.pallas_call(
    _scaleh_body,
    grid=(_GRID,),
    in_specs=[_degp_spec, _part_spec],
    out_specs=(_node_spec, _node_spec),
    out_shape=(jax.ShapeDtypeStruct((NPAD, D), jnp.float32),
               jax.ShapeDtypeStruct((NPAD, D), jnp.float32)),
)

_final = pl.pallas_call(
    _final_body,
    grid=(_GRID,),
    in_specs=[_degp_spec, _node_spec, _node_spec, _node_spec, _part_spec,
              _full((NHOPS + 1, D, D)), _full((8, D)),
              _full((D, D)), _full((8, D))],
    out_specs=_node_spec,
    out_shape=jax.ShapeDtypeStruct((NPAD, D), jnp.float32),
)


def kernel(x, edge_index, W, b, Wr, br):
    row = edge_index[0]
    col = edge_index[1]

    pad = EPAD - E
    ar = jnp.arange(pad, dtype=jnp.int32)
    rowp = jnp.concatenate([row, N + ar % (NACC - N)]).reshape(NW, NB, EB)
    colp = jnp.concatenate([col, ar % N]).reshape(NW, NB, EB)
    pk = (rowp << PSH) | colp

    xp = jnp.pad(x, ((0, NPAD - N), (0, 0)))
    b8 = jnp.pad(b, ((0, 8 - (NHOPS + 1)), (0, 0)))
    br8 = jnp.pad(br[None, :], ((0, 7), (0, 0)))

    degp = _deg_call(rowp)
    g = _scale0(degp, xp)
    feats = []
    for hop in range(NHOPS):
        p = _hop_call(pk, g)
        if hop < NHOPS - 1:
            h, g = _scaleh(degp, p)
            feats.append(h)
    out = _final(degp, xp, feats[0], feats[1], p, W, b8, Wr, br8)
    return out[:N]
